# baseline (speedup 1.0000x reference)
import dataclasses
import functools

import jax
import jax.numpy as jnp
from jax.experimental import pallas as pl
from jax.experimental.pallas import tpu as pltpu
from jax.experimental.pallas import tpu_sc as plsc

_B = 16384
_K = 1024
_R = 16


def _sc_permute(x, perm):
    mesh = plsc.VectorSubcoreMesh(core_axis_name="c", subcore_axis_name="s")
    cp = pltpu.CompilerParams()
    if "needs_layout_passes" in pltpu.CompilerParams.__dataclass_fields__:
        cp = dataclasses.replace(cp, needs_layout_passes=False)

    @functools.partial(
        pl.kernel,
        mesh=mesh,
        out_type=jax.ShapeDtypeStruct((_B, _K), jnp.float32),
        scratch_types=[pltpu.VMEM((_K,), jnp.int32)],
        compiler_params=cp,
    )
    def run(x_hbm, perm_hbm, out_hbm, idx_v):
        pltpu.sync_copy(perm_hbm, idx_v)

        def body(in_v, out_v):
            @pl.loop(0, _K // 16)
            def _(kc):
                col = idx_v[pl.ds(kc * 16, 16)]

                for r in range(_R):
                    row = jnp.full((16,), r, jnp.int32)
                    out_v[r, pl.ds(kc * 16, 16)] = plsc.load_gather(
                        in_v, [row, col]
                    )

        pltpu.emit_pipeline(
            body,
            grid=(_B // _R,),
            in_specs=[pl.BlockSpec((_R, _K), lambda i: (i, 0))],
            out_specs=[pl.BlockSpec((_R, _K), lambda i: (i, 0))],
            core_axis_name=("c", "s"),
            dimension_semantics=(pltpu.PARALLEL,),
        )(x_hbm, out_hbm)

    return run(x, perm)


def kernel(x, matrix):
    perm = jnp.transpose(matrix).reshape(_K).astype(jnp.int32)
    out = _sc_permute(x, perm)
    return out.reshape(_B, 32, 32)


# speedup vs baseline: 1.6047x; 1.6047x over previous
import dataclasses
import functools

import jax
import jax.numpy as jnp
from jax.experimental import pallas as pl
from jax.experimental.pallas import tpu as pltpu
from jax.experimental.pallas import tpu_sc as plsc

_B = 16384
_K = 1024
_R = 16


def _sc_permute(x, perm):
    mesh = plsc.VectorSubcoreMesh(core_axis_name="c", subcore_axis_name="s")
    cp = pltpu.CompilerParams()
    if "needs_layout_passes" in pltpu.CompilerParams.__dataclass_fields__:
        cp = dataclasses.replace(cp, needs_layout_passes=False)

    @functools.partial(
        pl.kernel,
        mesh=mesh,
        out_type=jax.ShapeDtypeStruct((_B, _K), jnp.float32),
        scratch_types=[pltpu.VMEM((_K,), jnp.int32)],
        compiler_params=cp,
    )
    def run(x_hbm, perm_hbm, out_hbm, idx_v):
        pltpu.sync_copy(perm_hbm, idx_v)

        def body(in_v, out_v):
            @pl.loop(0, _K // 16)
            def _(kc):
                col = idx_v[pl.ds(kc * 16, 16)]

                @plsc.parallel_loop(0, _R, 1, unroll=16)
                def _(r):
                    row = jnp.full((16,), r, jnp.int32)
                    out_v[r, pl.ds(kc * 16, 16)] = plsc.load_gather(
                        in_v, [row, col]
                    )

        pltpu.emit_pipeline(
            body,
            grid=(_B // _R,),
            in_specs=[pl.BlockSpec((_R, _K), lambda i: (i, 0))],
            out_specs=[pl.BlockSpec((_R, _K), lambda i: (i, 0))],
            core_axis_name=("c", "s"),
            dimension_semantics=(pltpu.PARALLEL,),
        )(x_hbm, out_hbm)

    return run(x, perm)


def kernel(x, matrix):
    perm = jnp.transpose(matrix).reshape(_K).astype(jnp.int32)
    out = _sc_permute(x, perm)
    return out.reshape(_B, 32, 32)


# speedup vs baseline: 1.6123x; 1.0047x over previous
import dataclasses
import functools

import jax
import jax.numpy as jnp
from jax import lax
from jax.experimental import pallas as pl
from jax.experimental.pallas import tpu as pltpu
from jax.experimental.pallas import tpu_sc as plsc

_B = 16384
_K = 1024
_R = 16
_NW = 32
_NB = _B // (_NW * _R)


def _sc_permute(x, perm):
    mesh = plsc.VectorSubcoreMesh(core_axis_name="c", subcore_axis_name="s")
    cp = pltpu.CompilerParams()
    if "needs_layout_passes" in pltpu.CompilerParams.__dataclass_fields__:
        cp = dataclasses.replace(cp, needs_layout_passes=False)

    @functools.partial(
        pl.kernel,
        mesh=mesh,
        out_type=jax.ShapeDtypeStruct((_B, _K), jnp.float32),
        scratch_types=[
            pltpu.VMEM((_K,), jnp.int32),
            pltpu.VMEM((_R, _K), jnp.float32),
            pltpu.VMEM((_R, _K), jnp.float32),
            pltpu.VMEM((_R, _K), jnp.float32),
            pltpu.VMEM((_R, _K), jnp.float32),
            pltpu.SemaphoreType.DMA,
            pltpu.SemaphoreType.DMA,
            pltpu.SemaphoreType.DMA,
            pltpu.SemaphoreType.DMA,
        ],
        compiler_params=cp,
    )
    def run(x_hbm, perm_hbm, out_hbm, idx_v,
            in0, in1, out0, out1, si0, si1, so0, so1):
        wid = lax.axis_index("s") * 2 + lax.axis_index("c")
        base = wid * (_NB * _R)
        pltpu.sync_copy(perm_hbm, idx_v)

        def rows(g):
            return pl.ds(base + g * _R, _R)

        def compute(in_v, out_v):
            @pl.loop(0, _K // 16)
            def _(kc):
                col = idx_v[pl.ds(kc * 16, 16)]

                @plsc.parallel_loop(0, _R, 1, unroll=16)
                def _(r):
                    row = jnp.full((16,), r, jnp.int32)
                    out_v[r, pl.ds(kc * 16, 16)] = plsc.load_gather(
                        in_v, [row, col]
                    )

        pltpu.async_copy(x_hbm.at[rows(0)], in0, si0)
        pltpu.async_copy(x_hbm.at[rows(1)], in1, si1)

        @pl.loop(0, _NB // 2)
        def _(it):
            g = it * 2
            for b, inb, outb, sib, sob in (
                (0, in0, out0, si0, so0),
                (1, in1, out1, si1, so1),
            ):
                pltpu.make_async_copy(x_hbm.at[rows(0)], inb, sib).wait()

                @pl.when(it > 0)
                def _():
                    pltpu.make_async_copy(outb, out_hbm.at[rows(0)], sob).wait()

                compute(inb, outb)
                pltpu.async_copy(outb, out_hbm.at[rows(g + b)], sob)

                @pl.when(it < _NB // 2 - 1)
                def _():
                    pltpu.async_copy(x_hbm.at[rows(g + b + 2)], inb, sib)

        pltpu.make_async_copy(out0, out_hbm.at[rows(0)], so0).wait()
        pltpu.make_async_copy(out1, out_hbm.at[rows(1)], so1).wait()

    return run(x, perm)


def kernel(x, matrix):
    perm = jnp.transpose(matrix).reshape(_K).astype(jnp.int32)
    out = _sc_permute(x, perm)
    return out.reshape(_B, 32, 32)
